# baseline (speedup 1.0000x reference)
import functools

import jax
import jax.numpy as jnp
from jax import lax
from jax.experimental import pallas as pl
from jax.experimental.pallas import tpu as pltpu
from jax.experimental.pallas import tpu_sc as plsc

N = 10000
E = 320000
D = 128

NC = 2
NS = 16
NW = NC * NS
EPW = E // NW
K = 125
CH = EPW // K
NH = 2
HCH = CH // NH
NP = 10240
RPT = NP // NS
CNT_PAD = NP
CPT = CNT_PAD // NS


def _make_segsum_body(with_cnt):
    def body(*refs):
        if with_cnt:
            (v_hbm, src_hbm, dst_hbm, zrows_hbm, ones_hbm, zcnt_hbm,
             acc_out, cnt_out,
             acc_sh, cnt_sh, src_v, dst_v, rows_a, rows_b, ones_v,
             cstage_v, sem_a, sem_b) = refs
        else:
            (v_hbm, src_hbm, dst_hbm, zrows_hbm,
             acc_out,
             acc_sh, src_v, dst_v, rows_a, rows_b, sem_a, sem_b) = refs
        cid = lax.axis_index("c")
        sid = lax.axis_index("s")
        wid = sid * NC + cid

        pltpu.sync_copy(zrows_hbm, rows_a)
        if with_cnt:
            pltpu.sync_copy(ones_hbm, ones_v)
            pltpu.sync_copy(zcnt_hbm, cstage_v)

        for r in range(RPT // 80):
            pltpu.sync_copy(rows_a.at[pl.ds(0, 80)],
                            acc_sh.at[pl.ds(sid * RPT + r * 80, 80)])
        if with_cnt:
            pltpu.sync_copy(cstage_v, cnt_sh.at[pl.ds(sid * CPT, CPT)])
        plsc.subcore_barrier()

        def gather(c, buf, sem):
            return pltpu.make_async_copy(v_hbm.at[src_v.at[c]], buf, sem)

        def scatter(c, buf):
            pltpu.sync_copy(buf, acc_sh.at[dst_v.at[c]], add=True)
            if with_cnt:
                pltpu.sync_copy(ones_v, cnt_sh.at[dst_v.at[c]], add=True)

        for half in range(NH):
            pltpu.sync_copy(src_hbm.at[wid, half], src_v)
            pltpu.sync_copy(dst_hbm.at[wid, half], dst_v)

            gather(0, rows_a, sem_a).start()

            def step(g, carry):
                c0 = 2 * g
                gather(c0 + 1, rows_b, sem_b).start()
                gather(c0, rows_a, sem_a).wait()
                scatter(c0, rows_a)

                @pl.when(g < HCH // 2 - 1)
                def _():
                    gather(c0 + 2, rows_a, sem_a).start()

                gather(c0 + 1, rows_b, sem_b).wait()
                scatter(c0 + 1, rows_b)
                return carry

            lax.fori_loop(0, HCH // 2, step, 0)
        plsc.subcore_barrier()

        for r in range(RPT // 80):
            base = sid * RPT + r * 80
            pltpu.sync_copy(acc_sh.at[pl.ds(base, 80)],
                            rows_a.at[pl.ds(0, 80)])
            pltpu.sync_copy(rows_a.at[pl.ds(0, 80)],
                            acc_out.at[cid, pl.ds(base, 80)])
        if with_cnt:
            pltpu.sync_copy(cnt_sh.at[pl.ds(sid * CPT, CPT)], cstage_v)
            pltpu.sync_copy(cstage_v, cnt_out.at[cid, pl.ds(sid * CPT, CPT)])

    return body


@functools.cache
def _get_segsum(with_cnt):
    if with_cnt:
        out_type = (
            jax.ShapeDtypeStruct((NC, NP, D), jnp.float32),
            jax.ShapeDtypeStruct((NC, CNT_PAD), jnp.float32),
        )
    else:
        out_type = jax.ShapeDtypeStruct((NC, NP, D), jnp.float32)
    scratch = [
        pltpu.VMEM_SHARED((NP, D), jnp.float32),
    ]
    if with_cnt:
        scratch.append(pltpu.VMEM_SHARED((CNT_PAD,), jnp.float32))
    scratch += [
        pltpu.VMEM((HCH, K), jnp.int32),
        pltpu.VMEM((HCH, K), jnp.int32),
        pltpu.VMEM((K, D), jnp.float32),
        pltpu.VMEM((K, D), jnp.float32),
    ]
    if with_cnt:
        scratch += [
            pltpu.VMEM((K,), jnp.float32),
            pltpu.VMEM((CPT,), jnp.float32),
        ]
    scratch += [pltpu.SemaphoreType.DMA, pltpu.SemaphoreType.DMA]
    return pl.kernel(
        _make_segsum_body(with_cnt),
        out_type=out_type,
        mesh=plsc.VectorSubcoreMesh(core_axis_name="c", subcore_axis_name="s",
                                    num_cores=NC, num_subcores=NS),
        scratch_types=scratch,
    )


B = 2000


def _tc_pre_body(v_ref, w_ref, b_ref, out_ref):
    out_ref[...] = (
        jnp.dot(v_ref[...], w_ref[...], preferred_element_type=jnp.float32)
        + b_ref[...]
    )


def _tc_post_body(acc_ref, cnt_ref, pre_ref, wl_ref, out_ref):
    c = cnt_ref[:, 0] + cnt_ref[:, 1]
    inv = 1.0 / jnp.maximum(c, 1.0)
    s = (acc_ref[0] + acc_ref[1]) * inv[:, None]
    out_ref[...] = jnp.maximum(
        jnp.dot(s, wl_ref[...], preferred_element_type=jnp.float32)
        + pre_ref[...],
        0.0,
    )


def _tc_post_final_body(acc_ref, cnt_ref, pre_ref, wl_ref,
                        wlin_ref, blin_ref, out_ref):
    c = cnt_ref[:, 0] + cnt_ref[:, 1]
    inv = 1.0 / jnp.maximum(c, 1.0)
    s = (acc_ref[0] + acc_ref[1]) * inv[:, None]
    z = jnp.maximum(
        jnp.dot(s, wl_ref[...], preferred_element_type=jnp.float32)
        + pre_ref[...],
        0.0,
    )
    out_ref[...] = (
        jnp.dot(z, wlin_ref[...], preferred_element_type=jnp.float32)
        + blin_ref[...]
    )


_W_SPEC = pl.BlockSpec((D, D), lambda i: (0, 0))
_B_SPEC = pl.BlockSpec((1, D), lambda i: (0, 0))
_ROW_SPEC = pl.BlockSpec((B, D), lambda i: (i, 0))
_POST_SPECS = [
    pl.BlockSpec((NC, B, D), lambda i: (0, i, 0)),
    pl.BlockSpec((B, NC), lambda i: (i, 0)),
    _ROW_SPEC,
    _W_SPEC,
]

_tc_pre = pl.pallas_call(
    _tc_pre_body,
    grid=(N // B,),
    in_specs=[_ROW_SPEC, _W_SPEC, _B_SPEC],
    out_specs=_ROW_SPEC,
    out_shape=jax.ShapeDtypeStruct((N, D), jnp.float32),
)

_tc_post = pl.pallas_call(
    _tc_post_body,
    grid=(N // B,),
    in_specs=_POST_SPECS,
    out_specs=_ROW_SPEC,
    out_shape=jax.ShapeDtypeStruct((N, D), jnp.float32),
)

_tc_post_final = pl.pallas_call(
    _tc_post_final_body,
    grid=(N // B,),
    in_specs=_POST_SPECS + [_W_SPEC, _B_SPEC],
    out_specs=_ROW_SPEC,
    out_shape=jax.ShapeDtypeStruct((N, D), jnp.float32),
)


def kernel(x, edge_index, Wl1, bl1, Wr1, Wl2, bl2, Wr2, Wlin, blin):
    src3 = edge_index[0].reshape(NW, NH, HCH, K)
    dst3 = edge_index[1].reshape(NW, NH, HCH, K)
    zrows = jnp.zeros((K, D), jnp.float32)
    ones = jnp.ones((K,), jnp.float32)
    zcnt = jnp.zeros((CPT,), jnp.float32)

    pre1 = _tc_pre(x, Wr1.T, bl1[None, :])
    acc1, cnt = _get_segsum(True)(x, src3, dst3, zrows, ones, zcnt)
    cnt_t = cnt.T
    h = _tc_post(acc1, cnt_t, pre1, Wl1.T)
    pre2 = _tc_pre(h, Wr2.T, bl2[None, :])
    acc2 = _get_segsum(False)(h, src3, dst3, zrows)
    out = _tc_post_final(acc2, cnt_t, pre2, Wl2.T, Wlin.T, blin[None, :])
    return out


# speedup vs baseline: 1.0575x; 1.0575x over previous
import functools

import jax
import jax.numpy as jnp
from jax import lax
from jax.experimental import pallas as pl
from jax.experimental.pallas import tpu as pltpu
from jax.experimental.pallas import tpu_sc as plsc

N = 10000
E = 320000
D = 128

NC = 2
NS = 16
NW = NC * NS
EPW = E // NW
K = 125
CH = EPW // K
NH = 2
HCH = CH // NH
NP = 10240
RPT = NP // NS
CNT_PAD = NP
CPT = CNT_PAD // NS


def _make_segsum_body(with_cnt):
    def body(*refs):
        if with_cnt:
            (v_hbm, ei_hbm, zrows_hbm, ones_hbm, zcnt_hbm,
             acc_out, cnt_out,
             acc_sh, cnt_sh, src_v, dst_v, rows_a, rows_b, ones_v,
             cstage_v, sem_a, sem_b) = refs
        else:
            (v_hbm, ei_hbm, zrows_hbm,
             acc_out,
             acc_sh, src_v, dst_v, rows_a, rows_b, sem_a, sem_b) = refs
        cid = lax.axis_index("c")
        sid = lax.axis_index("s")
        wid = sid * NC + cid

        pltpu.sync_copy(zrows_hbm, rows_a)
        if with_cnt:
            pltpu.sync_copy(ones_hbm, ones_v)
            pltpu.sync_copy(zcnt_hbm, cstage_v)

        for r in range(RPT // 80):
            pltpu.sync_copy(rows_a.at[pl.ds(0, 80)],
                            acc_sh.at[pl.ds(sid * RPT + r * 80, 80)])
        if with_cnt:
            pltpu.sync_copy(cstage_v, cnt_sh.at[pl.ds(sid * CPT, CPT)])
        plsc.subcore_barrier()

        def gather(c, buf, sem):
            return pltpu.make_async_copy(v_hbm.at[src_v.at[c]], buf, sem)

        def scatter(c, buf):
            pltpu.sync_copy(buf, acc_sh.at[dst_v.at[c]], add=True)
            if with_cnt:
                pltpu.sync_copy(ones_v, cnt_sh.at[dst_v.at[c]], add=True)

        for half in range(NH):
            pltpu.sync_copy(ei_hbm.at[0, wid, half], src_v)
            pltpu.sync_copy(ei_hbm.at[1, wid, half], dst_v)

            gather(0, rows_a, sem_a).start()

            def step(g, carry):
                c0 = 2 * g
                gather(c0 + 1, rows_b, sem_b).start()
                gather(c0, rows_a, sem_a).wait()
                scatter(c0, rows_a)

                @pl.when(g < HCH // 2 - 1)
                def _():
                    gather(c0 + 2, rows_a, sem_a).start()

                gather(c0 + 1, rows_b, sem_b).wait()
                scatter(c0 + 1, rows_b)
                return carry

            lax.fori_loop(0, HCH // 2, step, 0)
        plsc.subcore_barrier()

        for r in range(RPT // 80):
            base = sid * RPT + r * 80
            pltpu.sync_copy(acc_sh.at[pl.ds(base, 80)],
                            rows_a.at[pl.ds(0, 80)])
            pltpu.sync_copy(rows_a.at[pl.ds(0, 80)],
                            acc_out.at[cid, pl.ds(base, 80)])
        if with_cnt:
            pltpu.sync_copy(cnt_sh.at[pl.ds(sid * CPT, CPT)], cstage_v)
            pltpu.sync_copy(cstage_v, cnt_out.at[cid, pl.ds(sid * CPT, CPT)])

    return body


@functools.cache
def _get_segsum(with_cnt):
    if with_cnt:
        out_type = (
            jax.ShapeDtypeStruct((NC, NP, D), jnp.float32),
            jax.ShapeDtypeStruct((NC, CNT_PAD), jnp.float32),
        )
    else:
        out_type = jax.ShapeDtypeStruct((NC, NP, D), jnp.float32)
    scratch = [
        pltpu.VMEM_SHARED((NP, D), jnp.float32),
    ]
    if with_cnt:
        scratch.append(pltpu.VMEM_SHARED((CNT_PAD,), jnp.float32))
    scratch += [
        pltpu.VMEM((HCH, K), jnp.int32),
        pltpu.VMEM((HCH, K), jnp.int32),
        pltpu.VMEM((K, D), jnp.float32),
        pltpu.VMEM((K, D), jnp.float32),
    ]
    if with_cnt:
        scratch += [
            pltpu.VMEM((K,), jnp.float32),
            pltpu.VMEM((CPT,), jnp.float32),
        ]
    scratch += [pltpu.SemaphoreType.DMA, pltpu.SemaphoreType.DMA]
    return pl.kernel(
        _make_segsum_body(with_cnt),
        out_type=out_type,
        mesh=plsc.VectorSubcoreMesh(core_axis_name="c", subcore_axis_name="s",
                                    num_cores=NC, num_subcores=NS),
        scratch_types=scratch,
    )


B = 2000


def _mm_t(a, w):
    return lax.dot_general(a, w, (((1,), (1,)), ((), ())),
                           preferred_element_type=jnp.float32)


def _tc_layer_body(acc_ref, cnt_ref, v_ref, wl_ref, wr_ref, bl_ref, out_ref):
    c = cnt_ref[:, 0] + cnt_ref[:, 1]
    inv = 1.0 / jnp.maximum(c, 1.0)
    s = (acc_ref[0] + acc_ref[1]) * inv[:, None]
    out_ref[...] = jnp.maximum(
        _mm_t(s, wl_ref[...]) + _mm_t(v_ref[...], wr_ref[...]) + bl_ref[...],
        0.0,
    )


def _tc_final_body(acc_ref, cnt_ref, v_ref, wl_ref, wr_ref, bl_ref,
                   wlin_ref, blin_ref, out_ref):
    c = cnt_ref[:, 0] + cnt_ref[:, 1]
    inv = 1.0 / jnp.maximum(c, 1.0)
    s = (acc_ref[0] + acc_ref[1]) * inv[:, None]
    z = jnp.maximum(
        _mm_t(s, wl_ref[...]) + _mm_t(v_ref[...], wr_ref[...]) + bl_ref[...],
        0.0,
    )
    out_ref[...] = _mm_t(z, wlin_ref[...]) + blin_ref[...]


_W_SPEC = pl.BlockSpec((D, D), lambda i: (0, 0))
_B_SPEC = pl.BlockSpec((1, D), lambda i: (0, 0))
_ROW_SPEC = pl.BlockSpec((B, D), lambda i: (i, 0))
_LAYER_SPECS = [
    pl.BlockSpec((NC, B, D), lambda i: (0, i, 0)),
    pl.BlockSpec((B, NC), lambda i: (i, 0)),
    _ROW_SPEC,
    _W_SPEC,
    _W_SPEC,
    _B_SPEC,
]

_tc_layer = pl.pallas_call(
    _tc_layer_body,
    grid=(N // B,),
    in_specs=_LAYER_SPECS,
    out_specs=_ROW_SPEC,
    out_shape=jax.ShapeDtypeStruct((N, D), jnp.float32),
)

_tc_final = pl.pallas_call(
    _tc_final_body,
    grid=(N // B,),
    in_specs=_LAYER_SPECS + [_W_SPEC, _B_SPEC],
    out_specs=_ROW_SPEC,
    out_shape=jax.ShapeDtypeStruct((N, D), jnp.float32),
)


def kernel(x, edge_index, Wl1, bl1, Wr1, Wl2, bl2, Wr2, Wlin, blin):
    ei4 = edge_index.reshape(2, NW, NH, HCH, K)
    zrows = jnp.zeros((K, D), jnp.float32)
    ones = jnp.ones((K,), jnp.float32)
    zcnt = jnp.zeros((CPT,), jnp.float32)

    acc1, cnt = _get_segsum(True)(x, ei4, zrows, ones, zcnt)
    cnt_t = cnt.T
    h = _tc_layer(acc1, cnt_t, x, Wl1, Wr1, bl1[None, :])
    acc2 = _get_segsum(False)(h, ei4, zrows)
    out = _tc_final(acc2, cnt_t, h, Wl2, Wr2, bl2[None, :],
                    Wlin, blin[None, :])
    return out


# speedup vs baseline: 1.0593x; 1.0017x over previous
import functools

import jax
import jax.numpy as jnp
from jax import lax
from jax.experimental import pallas as pl
from jax.experimental.pallas import tpu as pltpu
from jax.experimental.pallas import tpu_sc as plsc

N = 10000
E = 320000
D = 128

NC = 2
NS = 16
NW = NC * NS
EPW = E // NW
K = 125
CH = EPW // K
NH = 2
HCH = CH // NH
NP = 10240
RPT = NP // NS
CNT_PAD = NP
CPT = CNT_PAD // NS


def _make_segsum_body(with_cnt):
    def body(*refs):
        if with_cnt:
            (v_hbm, ei_hbm, zacc_hbm, ones_hbm, zcnt_hbm,
             acc_out, cnt_out,
             acc_sh, cnt_sh, src_v, dst_v, rows_a, rows_b, ones_v,
             sem_a, sem_b) = refs
        else:
            (v_hbm, ei_hbm, zacc_hbm,
             acc_out,
             acc_sh, src_v, dst_v, rows_a, rows_b, sem_a, sem_b) = refs
        cid = lax.axis_index("c")
        sid = lax.axis_index("s")
        wid = sid * NC + cid

        pltpu.sync_copy(zacc_hbm, acc_sh.at[pl.ds(sid * RPT, RPT)])
        if with_cnt:
            pltpu.sync_copy(ones_hbm, ones_v)
            pltpu.sync_copy(zcnt_hbm, cnt_sh.at[pl.ds(sid * CPT, CPT)])
        plsc.subcore_barrier()

        def gather(c, buf, sem):
            return pltpu.make_async_copy(v_hbm.at[src_v.at[c]], buf, sem)

        def scatter(c, buf):
            pltpu.sync_copy(buf, acc_sh.at[dst_v.at[c]], add=True)
            if with_cnt:
                pltpu.sync_copy(ones_v, cnt_sh.at[dst_v.at[c]], add=True)

        for half in range(NH):
            pltpu.sync_copy(ei_hbm.at[0, wid, half], src_v)
            pltpu.sync_copy(ei_hbm.at[1, wid, half], dst_v)

            gather(0, rows_a, sem_a).start()

            def step(g, carry):
                c0 = 2 * g
                gather(c0 + 1, rows_b, sem_b).start()
                gather(c0, rows_a, sem_a).wait()
                scatter(c0, rows_a)

                @pl.when(g < HCH // 2 - 1)
                def _():
                    gather(c0 + 2, rows_a, sem_a).start()

                gather(c0 + 1, rows_b, sem_b).wait()
                scatter(c0 + 1, rows_b)
                return carry

            lax.fori_loop(0, HCH // 2, step, 0)
        plsc.subcore_barrier()

        pltpu.sync_copy(acc_sh.at[pl.ds(sid * RPT, RPT)],
                        acc_out.at[cid, pl.ds(sid * RPT, RPT)])
        if with_cnt:
            pltpu.sync_copy(cnt_sh.at[pl.ds(sid * CPT, CPT)],
                            cnt_out.at[cid, pl.ds(sid * CPT, CPT)])

    return body


@functools.cache
def _get_segsum(with_cnt):
    if with_cnt:
        out_type = (
            jax.ShapeDtypeStruct((NC, NP, D), jnp.float32),
            jax.ShapeDtypeStruct((NC, CNT_PAD), jnp.float32),
        )
    else:
        out_type = jax.ShapeDtypeStruct((NC, NP, D), jnp.float32)
    scratch = [
        pltpu.VMEM_SHARED((NP, D), jnp.float32),
    ]
    if with_cnt:
        scratch.append(pltpu.VMEM_SHARED((CNT_PAD,), jnp.float32))
    scratch += [
        pltpu.VMEM((HCH, K), jnp.int32),
        pltpu.VMEM((HCH, K), jnp.int32),
        pltpu.VMEM((K, D), jnp.float32),
        pltpu.VMEM((K, D), jnp.float32),
    ]
    if with_cnt:
        scratch += [
            pltpu.VMEM((K,), jnp.float32),
        ]
    scratch += [pltpu.SemaphoreType.DMA, pltpu.SemaphoreType.DMA]
    return pl.kernel(
        _make_segsum_body(with_cnt),
        out_type=out_type,
        mesh=plsc.VectorSubcoreMesh(core_axis_name="c", subcore_axis_name="s",
                                    num_cores=NC, num_subcores=NS),
        scratch_types=scratch,
    )


B = 2000


def _mm_t(a, w):
    return lax.dot_general(a, w, (((1,), (1,)), ((), ())),
                           preferred_element_type=jnp.float32)


def _tc_layer_body(acc_ref, cnt_ref, v_ref, wl_ref, wr_ref, bl_ref, out_ref):
    c = cnt_ref[:, 0] + cnt_ref[:, 1]
    inv = 1.0 / jnp.maximum(c, 1.0)
    s = (acc_ref[0] + acc_ref[1]) * inv[:, None]
    out_ref[...] = jnp.maximum(
        _mm_t(s, wl_ref[...]) + _mm_t(v_ref[...], wr_ref[...]) + bl_ref[...],
        0.0,
    )


def _tc_final_body(acc_ref, cnt_ref, v_ref, wl_ref, wr_ref, bl_ref,
                   wlin_ref, blin_ref, out_ref):
    c = cnt_ref[:, 0] + cnt_ref[:, 1]
    inv = 1.0 / jnp.maximum(c, 1.0)
    s = (acc_ref[0] + acc_ref[1]) * inv[:, None]
    z = jnp.maximum(
        _mm_t(s, wl_ref[...]) + _mm_t(v_ref[...], wr_ref[...]) + bl_ref[...],
        0.0,
    )
    out_ref[...] = _mm_t(z, wlin_ref[...]) + blin_ref[...]


_W_SPEC = pl.BlockSpec((D, D), lambda i: (0, 0))
_B_SPEC = pl.BlockSpec((1, D), lambda i: (0, 0))
_ROW_SPEC = pl.BlockSpec((B, D), lambda i: (i, 0))
_LAYER_SPECS = [
    pl.BlockSpec((NC, B, D), lambda i: (0, i, 0)),
    pl.BlockSpec((B, NC), lambda i: (i, 0)),
    _ROW_SPEC,
    _W_SPEC,
    _W_SPEC,
    _B_SPEC,
]

_tc_layer = pl.pallas_call(
    _tc_layer_body,
    grid=(N // B,),
    in_specs=_LAYER_SPECS,
    out_specs=_ROW_SPEC,
    out_shape=jax.ShapeDtypeStruct((N, D), jnp.float32),
)

_tc_final = pl.pallas_call(
    _tc_final_body,
    grid=(N // B,),
    in_specs=_LAYER_SPECS + [_W_SPEC, _B_SPEC],
    out_specs=_ROW_SPEC,
    out_shape=jax.ShapeDtypeStruct((N, D), jnp.float32),
)


def kernel(x, edge_index, Wl1, bl1, Wr1, Wl2, bl2, Wr2, Wlin, blin):
    ei4 = edge_index.reshape(2, NW, NH, HCH, K)
    zacc = jnp.zeros((RPT, D), jnp.float32)
    ones = jnp.ones((K,), jnp.float32)
    zcnt = jnp.zeros((CPT,), jnp.float32)

    acc1, cnt = _get_segsum(True)(x, ei4, zacc, ones, zcnt)
    cnt_t = cnt.T
    h = _tc_layer(acc1, cnt_t, x, Wl1, Wr1, bl1[None, :])
    acc2 = _get_segsum(False)(h, ei4, zacc)
    out = _tc_final(acc2, cnt_t, h, Wl2, Wr2, bl2[None, :],
                    Wlin, blin[None, :])
    return out
